# Initial kernel scaffold; baseline (speedup 1.0000x reference)
import jax
import jax.numpy as jnp
from jax.experimental import pallas as pl


def kernel(x, router_w, egate_w, eup_w, edown_w, sh_gate_w, sh_up_w, sh_down_w, shg_w, shg_b):
    raise NotImplementedError("write your pallas kernel here")


# speedup vs baseline: 1.2612x; 1.2612x over previous
import functools

import jax
import jax.numpy as jnp
from jax import lax
from jax.experimental import pallas as pl
from jax.experimental.pallas import tpu as pltpu
from jax.experimental.pallas import tpu_sc as plsc

_TILE = 256


def _router_argmax(x2d, router_w):
    N, D = x2d.shape
    E = router_w.shape[0]
    TB = 1024

    def body(x_ref, w_ref, o_ref):
        logits = lax.dot_general(
            x_ref[...].astype(jnp.bfloat16), w_ref[...].astype(jnp.bfloat16),
            (((1,), (1,)), ((), ())),
            preferred_element_type=jnp.float32)
        maxv = jnp.max(logits, axis=1, keepdims=True)
        ids = lax.broadcasted_iota(jnp.int32, logits.shape, 1)
        o_ref[...] = jnp.min(jnp.where(logits >= maxv, ids, E),
                             axis=1, keepdims=True)

    out = pl.pallas_call(
        body,
        grid=(N // TB,),
        in_specs=[pl.BlockSpec((TB, D), lambda i: (i, 0)),
                  pl.BlockSpec((E, D), lambda i: (0, 0))],
        out_specs=pl.BlockSpec((TB, 1), lambda i: (i, 0)),
        out_shape=jax.ShapeDtypeStruct((N, 1), jnp.int32),
    )(x2d, router_w)
    return out[:, 0]


def _build_dispatch(e_idx, E, T, NPAD):
    N = e_idx.shape[0]
    onehot = (e_idx[:, None] == jnp.arange(E, dtype=e_idx.dtype)[None, :]
              ).astype(jnp.int32)
    ranks = jnp.cumsum(onehot, axis=0) - 1
    rank_t = jnp.take_along_axis(ranks, e_idx[:, None], axis=1)[:, 0]
    counts = jnp.sum(onehot, axis=0)
    padded = ((counts + T - 1) // T) * T
    p_end = jnp.cumsum(padded)
    p_off = p_end - padded
    dst = (p_off[e_idx] + rank_t).astype(jnp.int32)
    src = jnp.zeros((NPAD,), jnp.int32).at[dst].set(
        jnp.arange(N, dtype=jnp.int32))
    nt = NPAD // T
    tile_expert = jnp.searchsorted(
        p_end, jnp.arange(nt, dtype=p_end.dtype) * T, side='right')
    tile_expert = jnp.minimum(tile_expert, E - 1).astype(jnp.int32)
    return dst, src, tile_expert


def _sc_gather(table, idx):
    V, D = table.shape
    Bn = idx.shape[0]
    info = plsc.get_sparse_core_info()
    NC = info.num_cores
    NW = NC * info.num_subcores
    bpw = Bn // NW
    CH = 64
    mesh = plsc.VectorSubcoreMesh(core_axis_name="c", subcore_axis_name="s")

    @functools.partial(
        pl.kernel, mesh=mesh,
        out_type=jax.ShapeDtypeStruct((Bn, D), table.dtype),
        scratch_types=[pltpu.VMEM((CH,), jnp.int32),
                       pltpu.VMEM((CH, D), table.dtype),
                       pltpu.SemaphoreType.DMA])
    def gk(table_hbm, idx_hbm, out_hbm, idx_v, rows_v, sem):
        wid = lax.axis_index("s") * NC + lax.axis_index("c")
        base = wid * bpw
        for c in range(bpw // CH):
            off = base + c * CH
            pltpu.sync_copy(idx_hbm.at[pl.ds(off, CH)], idx_v)
            pltpu.async_copy(table_hbm.at[idx_v], rows_v, sem).wait()
            pltpu.sync_copy(rows_v, out_hbm.at[pl.ds(off, CH)])

    return gk(table, idx)


def _grouped_swiglu(tile_expert, x_sorted, egate_bf, eup_bf, edown_bf):
    NPAD, D = x_sorted.shape
    E, F, _ = egate_bf.shape
    nt = NPAD // _TILE

    def body(te_ref, xs_ref, gw_ref, uw_ref, dw_ref, o_ref):
        xb = xs_ref[...].astype(jnp.bfloat16)
        g = lax.dot_general(xb, gw_ref[0], (((1,), (1,)), ((), ())),
                            preferred_element_type=jnp.float32)
        u = lax.dot_general(xb, uw_ref[0], (((1,), (1,)), ((), ())),
                            preferred_element_type=jnp.float32)
        h = (g * jax.nn.sigmoid(g) * u).astype(jnp.bfloat16)
        o_ref[...] = lax.dot_general(h, dw_ref[0], (((1,), (1,)), ((), ())),
                                     preferred_element_type=jnp.float32)

    grid_spec = pltpu.PrefetchScalarGridSpec(
        num_scalar_prefetch=1,
        grid=(nt,),
        in_specs=[pl.BlockSpec((_TILE, D), lambda i, te: (i, 0)),
                  pl.BlockSpec((1, F, D), lambda i, te: (te[i], 0, 0)),
                  pl.BlockSpec((1, F, D), lambda i, te: (te[i], 0, 0)),
                  pl.BlockSpec((1, D, F), lambda i, te: (te[i], 0, 0))],
        out_specs=pl.BlockSpec((_TILE, D), lambda i, te: (i, 0)),
    )
    return pl.pallas_call(
        body, grid_spec=grid_spec,
        out_shape=jax.ShapeDtypeStruct((NPAD, D), jnp.float32),
    )(tile_expert, x_sorted, egate_bf, eup_bf, edown_bf)


def _shared_combine(x2d, gw_bf, uw_bf, dw_bf, shg_w, shg_b2, routed):
    N, D = x2d.shape
    F = gw_bf.shape[0]
    TB = 256

    def body(x_ref, gw_ref, uw_ref, dw_ref, sg_ref, sb_ref, r_ref, y_ref):
        xf = x_ref[...]
        xb = xf.astype(jnp.bfloat16)
        g = lax.dot_general(xb, gw_ref[...], (((1,), (1,)), ((), ())),
                            preferred_element_type=jnp.float32)
        u = lax.dot_general(xb, uw_ref[...], (((1,), (1,)), ((), ())),
                            preferred_element_type=jnp.float32)
        h = (g * jax.nn.sigmoid(g) * u).astype(jnp.bfloat16)
        sh = lax.dot_general(h, dw_ref[...], (((1,), (1,)), ((), ())),
                             preferred_element_type=jnp.float32)
        logit = jnp.sum(xf * sg_ref[...], axis=1, keepdims=True)
        alpha = jax.nn.sigmoid(logit + sb_ref[0, 0])
        y_ref[...] = alpha * sh + (1.0 - alpha) * r_ref[...]

    return pl.pallas_call(
        body,
        grid=(N // TB,),
        in_specs=[pl.BlockSpec((TB, D), lambda i: (i, 0)),
                  pl.BlockSpec((F, D), lambda i: (0, 0)),
                  pl.BlockSpec((F, D), lambda i: (0, 0)),
                  pl.BlockSpec((D, F), lambda i: (0, 0)),
                  pl.BlockSpec((1, D), lambda i: (0, 0)),
                  pl.BlockSpec((1, 1), lambda i: (0, 0)),
                  pl.BlockSpec((TB, D), lambda i: (i, 0))],
        out_specs=pl.BlockSpec((TB, D), lambda i: (i, 0)),
        out_shape=jax.ShapeDtypeStruct((N, D), jnp.float32),
    )(x2d, gw_bf, uw_bf, dw_bf, shg_w, shg_b2, routed)


def kernel(x, router_w, egate_w, eup_w, edown_w,
           sh_gate_w, sh_up_w, sh_down_w, shg_w, shg_b):
    B, S, D = x.shape
    N = B * S
    E = router_w.shape[0]
    x2d = x.reshape(N, D)
    NPAD = N + E * _TILE

    e_idx = _router_argmax(x2d, router_w)
    dst, src, tile_expert = _build_dispatch(e_idx, E, _TILE, NPAD)

    x_sorted = _sc_gather(x2d, src)
    out_pad = _grouped_swiglu(tile_expert, x_sorted,
                              egate_w.astype(jnp.bfloat16),
                              eup_w.astype(jnp.bfloat16),
                              edown_w.astype(jnp.bfloat16))
    routed = _sc_gather(out_pad, dst)

    y2d = _shared_combine(x2d,
                          sh_gate_w.astype(jnp.bfloat16),
                          sh_up_w.astype(jnp.bfloat16),
                          sh_down_w.astype(jnp.bfloat16),
                          shg_w, shg_b.reshape(1, 1), routed)
    return y2d.reshape(B, S, D)
